# baseline (speedup 1.0000x reference)
import functools

import jax
import jax.numpy as jnp
from jax import lax
from jax.experimental import pallas as pl
from jax.experimental.pallas import tpu as pltpu
from jax.experimental.pallas import tpu_sc as plsc

B, L, D = 16, 2048, 128
NC, NS, LANES = 2, 16, 16
NW = NC * NS
RW = (B * L) // NW
CH = 128
NCH = RW // CH

_mesh = plsc.VectorSubcoreMesh(core_axis_name="c", subcore_axis_name="s")


@functools.partial(
    pl.kernel,
    out_type=jax.ShapeDtypeStruct((B * L, D), jnp.float32),
    mesh=_mesh,
    scratch_types=[
        pltpu.VMEM((L + LANES,), jnp.int32),
        pltpu.VMEM((NCH, CH), jnp.int32),
        pltpu.VMEM((6, CH, D), jnp.float32),
        pltpu.SemaphoreType.DMA,
        pltpu.SemaphoreType.DMA,
        pltpu.SemaphoreType.DMA,
        pltpu.SemaphoreType.DMA,
        pltpu.SemaphoreType.DMA,
        pltpu.SemaphoreType.DMA,
        pltpu.SemaphoreType.DMA,
        pltpu.SemaphoreType.DMA,
        pltpu.SemaphoreType.DMA,
        pltpu.SemaphoreType.DMA,
        pltpu.SemaphoreType.DMA,
        pltpu.SemaphoreType.DMA,
    ],
)
def _flatten_sc(mask_hbm, ex_hbm, out_ex,
                mask_v, idx_v, rows_v,
                gsem0, gsem1, gsem2, gsem3, gsem4, gsem5,
                wsem0, wsem1, wsem2, wsem3, wsem4, wsem5):
    NSLOT = 6
    gsem = (gsem0, gsem1, gsem2, gsem3, gsem4, gsem5)
    wsem = (wsem0, wsem1, wsem2, wsem3, wsem4, wsem5)
    wid = lax.axis_index("s") * NC + lax.axis_index("c")
    b = wid // 2
    half = wid % 2
    row0 = b * L + half * RW
    jbase = half * RW

    pltpu.sync_copy(mask_hbm.at[b], mask_v.at[pl.ds(0, L)])

    def _bs_body(_, carry):
        lo, hi = carry
        mid = (lo + hi) // 2
        go_right = mask_v[pl.ds(mid, LANES)][0] != 0
        return (jnp.where(go_right, mid + 1, lo),
                jnp.where(go_right, hi, mid))

    nv, _ = lax.fori_loop(0, 11, _bs_body,
                          (jnp.int32(0), jnp.int32(L)))

    lane = lax.iota(jnp.int32, 16)
    for c in range(NCH):
        for i in range(CH // LANES):
            j = jbase + c * CH + i * LANES + lane
            idx_v[c, pl.ds(i * LANES, LANES)] = b * L + lax.rem(j, nv)

    gathers = [None] * NCH
    writes = [None] * NCH
    for c in range(min(NSLOT, NCH)):
        gathers[c] = pltpu.async_copy(
            ex_hbm.at[idx_v.at[c]], rows_v.at[c % NSLOT], gsem[c % NSLOT])
    for c in range(NCH):
        s = c % NSLOT
        gathers[c].wait()
        writes[c] = pltpu.async_copy(
            rows_v.at[s], out_ex.at[pl.ds(row0 + c * CH, CH)], wsem[s])
        if c + NSLOT < NCH:
            writes[c].wait()
            gathers[c + NSLOT] = pltpu.async_copy(
                ex_hbm.at[idx_v.at[c + NSLOT]], rows_v.at[s], gsem[s])
    for c in range(max(0, NCH - NSLOT), NCH):
        writes[c].wait()


def _ctx_body(ctx_ref, out_ref):
    row = ctx_ref[pl.ds(pl.program_id(0), 1), :]
    out_ref[...] = jnp.broadcast_to(row, out_ref.shape)


_ctx_broadcast = pl.pallas_call(
    _ctx_body,
    grid=(B,),
    in_specs=[pl.BlockSpec((B, D), lambda i: (0, 0))],
    out_specs=pl.BlockSpec((L, D), lambda i: (i, 0)),
    out_shape=jax.ShapeDtypeStruct((B * L, D), jnp.float32),
)


def kernel(context_feature, example_feature, list_mask):
    mask_i32 = list_mask.astype(jnp.int32)
    ex2d = example_feature.reshape(B * L, D)
    flat_ex = _flatten_sc(mask_i32, ex2d)
    flat_ctx = _ctx_broadcast(context_feature)
    return flat_ctx, flat_ex


# speedup vs baseline: 1.0522x; 1.0522x over previous
import functools

import jax
import jax.numpy as jnp
from jax import lax
from jax.experimental import pallas as pl
from jax.experimental.pallas import tpu as pltpu
from jax.experimental.pallas import tpu_sc as plsc

B, L, D = 16, 2048, 128
NC, NS, LANES = 2, 16, 16
NW = NC * NS
RW = (B * L) // NW
CH = 128
NCH = RW // CH

_mesh = plsc.VectorSubcoreMesh(core_axis_name="c", subcore_axis_name="s")


@functools.partial(
    pl.kernel,
    out_type=jax.ShapeDtypeStruct((B * L, D), jnp.float32),
    mesh=_mesh,
    scratch_types=[
        pltpu.VMEM((L + LANES,), jnp.int32),
        pltpu.VMEM((NCH, CH), jnp.int32),
        pltpu.VMEM((4, CH, D), jnp.float32),
        pltpu.SemaphoreType.DMA,
        pltpu.SemaphoreType.DMA,
        pltpu.SemaphoreType.DMA,
        pltpu.SemaphoreType.DMA,
        pltpu.SemaphoreType.DMA,
        pltpu.SemaphoreType.DMA,
        pltpu.SemaphoreType.DMA,
        pltpu.SemaphoreType.DMA,
    ],
)
def _flatten_sc(mask_hbm, ex_hbm, out_ex,
                mask_v, idx_v, rows_v,
                gsem0, gsem1, gsem2, gsem3,
                wsem0, wsem1, wsem2, wsem3):
    NSLOT = 4
    gsem = (gsem0, gsem1, gsem2, gsem3)
    wsem = (wsem0, wsem1, wsem2, wsem3)
    wid = lax.axis_index("s") * NC + lax.axis_index("c")
    b = wid // 2
    half = wid % 2
    row0 = b * L + half * RW
    jbase = half * RW

    pltpu.sync_copy(mask_hbm.at[b], mask_v.at[pl.ds(0, L)])

    def _bs_body(_, carry):
        lo, hi = carry
        mid = (lo + hi) // 2
        go_right = mask_v[pl.ds(mid, LANES)][0] != 0
        return (jnp.where(go_right, mid + 1, lo),
                jnp.where(go_right, hi, mid))

    nv, _ = lax.fori_loop(0, 11, _bs_body,
                          (jnp.int32(0), jnp.int32(L)))

    lane = lax.iota(jnp.int32, 16)

    def _build(c):
        for i in range(CH // LANES):
            j = jbase + c * CH + i * LANES + lane
            idx_v[c, pl.ds(i * LANES, LANES)] = b * L + lax.rem(j, nv)

    gathers = [None] * NCH
    writes = [None] * NCH
    for c in range(min(NSLOT, NCH)):
        _build(c)
        gathers[c] = pltpu.async_copy(
            ex_hbm.at[idx_v.at[c]], rows_v.at[c % NSLOT], gsem[c % NSLOT])
    for c in range(min(NSLOT, NCH), NCH):
        _build(c)
    for c in range(NCH):
        s = c % NSLOT
        gathers[c].wait()
        writes[c] = pltpu.async_copy(
            rows_v.at[s], out_ex.at[pl.ds(row0 + c * CH, CH)], wsem[s])
        if c + NSLOT < NCH:
            writes[c].wait()
            gathers[c + NSLOT] = pltpu.async_copy(
                ex_hbm.at[idx_v.at[c + NSLOT]], rows_v.at[s], gsem[s])
    for c in range(max(0, NCH - NSLOT), NCH):
        writes[c].wait()


def _ctx_body(ctx_ref, out_ref):
    row = ctx_ref[pl.ds(pl.program_id(0), 1), :]
    out_ref[...] = jnp.broadcast_to(row, out_ref.shape)


_ctx_broadcast = pl.pallas_call(
    _ctx_body,
    grid=(B,),
    in_specs=[pl.BlockSpec((B, D), lambda i: (0, 0))],
    out_specs=pl.BlockSpec((L, D), lambda i: (i, 0)),
    out_shape=jax.ShapeDtypeStruct((B * L, D), jnp.float32),
)


def kernel(context_feature, example_feature, list_mask):
    mask_i32 = list_mask.astype(jnp.int32)
    ex2d = example_feature.reshape(B * L, D)
    flat_ex = _flatten_sc(mask_i32, ex2d)
    flat_ctx = _ctx_broadcast(context_feature)
    return flat_ctx, flat_ex


# speedup vs baseline: 1.0523x; 1.0001x over previous
import functools

import jax
import jax.numpy as jnp
from jax import lax
from jax.experimental import pallas as pl
from jax.experimental.pallas import tpu as pltpu
from jax.experimental.pallas import tpu_sc as plsc

B, L, D = 16, 2048, 128
NC, NS, LANES = 2, 16, 16
NW = NC * NS
RW = (B * L) // NW
CH = 128
NCH = RW // CH

_mesh = plsc.VectorSubcoreMesh(core_axis_name="c", subcore_axis_name="s")


@functools.partial(
    pl.kernel,
    out_type=jax.ShapeDtypeStruct((B * L, D), jnp.float32),
    mesh=_mesh,
    scratch_types=[
        pltpu.VMEM((L + LANES,), jnp.int32),
        pltpu.VMEM((NCH, CH), jnp.int32),
        pltpu.VMEM((4, CH, D), jnp.float32),
        pltpu.SemaphoreType.DMA,
        pltpu.SemaphoreType.DMA,
        pltpu.SemaphoreType.DMA,
        pltpu.SemaphoreType.DMA,
        pltpu.SemaphoreType.DMA,
        pltpu.SemaphoreType.DMA,
        pltpu.SemaphoreType.DMA,
        pltpu.SemaphoreType.DMA,
    ],
)
def _flatten_sc(mask_hbm, ex_hbm, out_ex,
                mask_v, idx_v, rows_v,
                gsem0, gsem1, gsem2, gsem3,
                wsem0, wsem1, wsem2, wsem3):
    NSLOT = 4
    gsem = (gsem0, gsem1, gsem2, gsem3)
    wsem = (wsem0, wsem1, wsem2, wsem3)
    wid = lax.axis_index("s") * NC + lax.axis_index("c")
    b = wid // 2
    half = wid % 2
    row0 = b * L + half * RW
    jbase = half * RW

    pltpu.sync_copy(mask_hbm.at[b], mask_v.at[pl.ds(0, L)])

    def _bs_body(_, carry):
        lo, hi = carry
        mid = (lo + hi) // 2
        go_right = mask_v[pl.ds(mid, LANES)][0] != 0
        return (jnp.where(go_right, mid + 1, lo),
                jnp.where(go_right, hi, mid))

    nv, _ = lax.fori_loop(0, 11, _bs_body,
                          (jnp.int32(0), jnp.int32(L)))

    lane = lax.iota(jnp.int32, 16)

    def _build(c):
        for i in range(CH // LANES):
            j = jbase + c * CH + i * LANES + lane
            idx_v[c, pl.ds(i * LANES, LANES)] = b * L + lax.rem(j, nv)

    def _fire(c, s):
        is_identity = (jbase + c * CH + CH) <= nv

        @pl.when(is_identity)
        def _():
            pltpu.async_copy(
                ex_hbm.at[pl.ds(row0 + c * CH, CH)], rows_v.at[s], gsem[s])

        @pl.when(jnp.logical_not(is_identity))
        def _():
            pltpu.async_copy(ex_hbm.at[idx_v.at[c]], rows_v.at[s], gsem[s])

    def _wait_gather(s):
        pltpu.make_async_copy(
            ex_hbm.at[pl.ds(0, CH)], rows_v.at[s], gsem[s]).wait()

    writes = [None] * NCH
    for c in range(min(NSLOT, NCH)):
        _build(c)
        _fire(c, c % NSLOT)
    for c in range(min(NSLOT, NCH), NCH):
        _build(c)
    for c in range(NCH):
        s = c % NSLOT
        _wait_gather(s)
        writes[c] = pltpu.async_copy(
            rows_v.at[s], out_ex.at[pl.ds(row0 + c * CH, CH)], wsem[s])
        if c + NSLOT < NCH:
            writes[c].wait()
            _fire(c + NSLOT, s)
    for c in range(max(0, NCH - NSLOT), NCH):
        writes[c].wait()


def _ctx_body(ctx_ref, out_ref):
    row = ctx_ref[pl.ds(pl.program_id(0), 1), :]
    out_ref[...] = jnp.broadcast_to(row, out_ref.shape)


_ctx_broadcast = pl.pallas_call(
    _ctx_body,
    grid=(B,),
    in_specs=[pl.BlockSpec((B, D), lambda i: (0, 0))],
    out_specs=pl.BlockSpec((L, D), lambda i: (i, 0)),
    out_shape=jax.ShapeDtypeStruct((B * L, D), jnp.float32),
)


def kernel(context_feature, example_feature, list_mask):
    mask_i32 = list_mask.astype(jnp.int32)
    ex2d = example_feature.reshape(B * L, D)
    flat_ex = _flatten_sc(mask_i32, ex2d)
    flat_ctx = _ctx_broadcast(context_feature)
    return flat_ctx, flat_ex
